# baseline (device time: 168313 ns/iter reference)
import jax
import jax.numpy as jnp
from jax import lax
from jax.experimental import pallas as pl
from jax.experimental.pallas import tpu as pltpu

N_DEV = 32


def _gelu(y):
    c = 0.7978845608028654
    return 0.5 * y * (1.0 + jnp.tanh(c * (y + 0.044715 * y * y * y)))


def kernel(x, w_mat):
    m, _ = x.shape
    _, n = w_mat.shape
    per = m // N_DEV
    nhops = N_DEV - 1

    def body(x_ref, w_ref, out_ref, acc, sbuf, rbuf, gbuf,
             s_rs, r_rs, s_ag, r_ag):
        my = lax.axis_index("i")
        left = lax.rem(my + (N_DEV - 1), N_DEV)
        right = lax.rem(my + 1, N_DEV)

        barrier = pltpu.get_barrier_semaphore()
        for nbr in (left, right):
            pl.semaphore_signal(barrier, inc=1, device_id=(nbr,),
                                device_id_type=pl.DeviceIdType.MESH)
        pl.semaphore_wait(barrier, 2)

        acc[...] = jnp.dot(x_ref[...], w_ref[...],
                           preferred_element_type=jnp.float32)

        for h in range(nhops):
            c_s = lax.rem(my - h + 2 * N_DEV, N_DEV)
            sbuf[h] = acc[pl.ds(c_s * per, per), :].astype(jnp.bfloat16)
            rdma = pltpu.make_async_remote_copy(
                src_ref=sbuf.at[h], dst_ref=rbuf.at[h],
                send_sem=s_rs.at[h], recv_sem=r_rs.at[h],
                device_id=(right,), device_id_type=pl.DeviceIdType.MESH)
            rdma.start()
            rdma.wait()
            c_r = lax.rem(my - h - 1 + 2 * N_DEV, N_DEV)
            acc[pl.ds(c_r * per, per), :] += rbuf[h].astype(jnp.float32)

        c_own = right
        g = _gelu(acc[pl.ds(c_own * per, per), :])
        out_ref[pl.ds(c_own * per, per), :] = g
        gbuf[0] = g.astype(jnp.bfloat16)

        for h in range(nhops):
            rdma = pltpu.make_async_remote_copy(
                src_ref=gbuf.at[h], dst_ref=gbuf.at[h + 1],
                send_sem=s_ag.at[h], recv_sem=r_ag.at[h],
                device_id=(right,), device_id_type=pl.DeviceIdType.MESH)
            rdma.start()
            rdma.wait()
            c = lax.rem(my - h + 2 * N_DEV, N_DEV)
            out_ref[pl.ds(c * per, per), :] = gbuf[h + 1].astype(jnp.float32)

    out_shape = jax.ShapeDtypeStruct((m, n), jnp.float32)
    return pl.pallas_call(
        body,
        out_shape=out_shape,
        in_specs=[pl.BlockSpec(memory_space=pltpu.VMEM),
                  pl.BlockSpec(memory_space=pltpu.VMEM)],
        out_specs=pl.BlockSpec(memory_space=pltpu.VMEM),
        scratch_shapes=[
            pltpu.VMEM((m, n), jnp.float32),
            pltpu.VMEM((nhops, per, n), jnp.bfloat16),
            pltpu.VMEM((nhops, per, n), jnp.bfloat16),
            pltpu.VMEM((N_DEV, per, n), jnp.bfloat16),
            pltpu.SemaphoreType.DMA((nhops,)),
            pltpu.SemaphoreType.DMA((nhops,)),
            pltpu.SemaphoreType.DMA((nhops,)),
            pltpu.SemaphoreType.DMA((nhops,)),
        ],
        compiler_params=pltpu.CompilerParams(collective_id=0),
    )(x, w_mat)


# device time: 65851 ns/iter; 2.5560x vs baseline; 2.5560x over previous
import jax
import jax.numpy as jnp
from jax import lax
from jax.experimental import pallas as pl
from jax.experimental.pallas import tpu as pltpu

N_DEV = 32


def _gelu(y):
    c = 0.7978845608028654
    return 0.5 * y * (1.0 + jnp.tanh(c * (y + 0.044715 * y * y * y)))


def kernel(x, w_mat):
    m, _ = x.shape
    _, n = w_mat.shape
    per = m // N_DEV

    def body(x_ref, w_ref, out_ref, part, sbuf, rbuf, gsend, obuf,
             s1, r1, s2, r2):
        my = lax.axis_index("i")

        barrier = pltpu.get_barrier_semaphore()
        for k in range(1, N_DEV):
            p = lax.rem(my + k, N_DEV)
            pl.semaphore_signal(barrier, inc=1, device_id=(p,),
                                device_id_type=pl.DeviceIdType.MESH)
        pl.semaphore_wait(barrier, N_DEV - 1)

        part[...] = jnp.dot(x_ref[...], w_ref[...],
                            preferred_element_type=jnp.float32)

        send1 = []
        for k in range(1, N_DEV):
            p = lax.rem(my + k, N_DEV)
            sbuf[k] = part[pl.ds(p * per, per), :].astype(jnp.bfloat16)
            rdma = pltpu.make_async_remote_copy(
                src_ref=sbuf.at[k], dst_ref=rbuf.at[N_DEV - k],
                send_sem=s1.at[k], recv_sem=r1.at[N_DEV - k],
                device_id=(p,), device_id_type=pl.DeviceIdType.MESH)
            rdma.start()
            send1.append(rdma)

        tot = part[pl.ds(my * per, per), :]
        for j in range(1, N_DEV):
            recv = pltpu.make_async_remote_copy(
                src_ref=sbuf.at[j], dst_ref=rbuf.at[j],
                send_sem=s1.at[j], recv_sem=r1.at[j],
                device_id=(my,), device_id_type=pl.DeviceIdType.MESH)
            recv.wait_recv()
            tot = tot + rbuf[j].astype(jnp.float32)

        g = _gelu(tot)
        out_ref[pl.ds(my * per, per), :] = g
        gsend[...] = g.astype(jnp.bfloat16)

        send2 = []
        for k in range(1, N_DEV):
            p = lax.rem(my + k, N_DEV)
            rdma = pltpu.make_async_remote_copy(
                src_ref=gsend, dst_ref=obuf.at[N_DEV - k],
                send_sem=s2.at[k], recv_sem=r2.at[N_DEV - k],
                device_id=(p,), device_id_type=pl.DeviceIdType.MESH)
            rdma.start()
            send2.append(rdma)

        for j in range(1, N_DEV):
            recv = pltpu.make_async_remote_copy(
                src_ref=gsend, dst_ref=obuf.at[j],
                send_sem=s2.at[j], recv_sem=r2.at[j],
                device_id=(my,), device_id_type=pl.DeviceIdType.MESH)
            recv.wait_recv()
            c = lax.rem(my + j, N_DEV)
            out_ref[pl.ds(c * per, per), :] = obuf[j].astype(jnp.float32)

        for rdma in send1 + send2:
            rdma.wait_send()

    out_shape = jax.ShapeDtypeStruct((m, n), jnp.float32)
    return pl.pallas_call(
        body,
        out_shape=out_shape,
        in_specs=[pl.BlockSpec(memory_space=pltpu.VMEM),
                  pl.BlockSpec(memory_space=pltpu.VMEM)],
        out_specs=pl.BlockSpec(memory_space=pltpu.VMEM),
        scratch_shapes=[
            pltpu.VMEM((m, n), jnp.float32),
            pltpu.VMEM((N_DEV, per, n), jnp.bfloat16),
            pltpu.VMEM((N_DEV, per, n), jnp.bfloat16),
            pltpu.VMEM((per, n), jnp.bfloat16),
            pltpu.VMEM((N_DEV, per, n), jnp.bfloat16),
            pltpu.SemaphoreType.DMA((N_DEV,)),
            pltpu.SemaphoreType.DMA((N_DEV,)),
            pltpu.SemaphoreType.DMA((N_DEV,)),
            pltpu.SemaphoreType.DMA((N_DEV,)),
        ],
        compiler_params=pltpu.CompilerParams(collective_id=0),
    )(x, w_mat)


# device time: 62608 ns/iter; 2.6884x vs baseline; 1.0518x over previous
import jax
import jax.numpy as jnp
from jax import lax
from jax.experimental import pallas as pl
from jax.experimental.pallas import tpu as pltpu

N_DEV = 32
W = 2


def _gelu(y):
    c = 0.7978845608028654
    return 0.5 * y * (1.0 + jnp.tanh(c * (y + 0.044715 * y * y * y)))


def kernel(x, w_mat):
    m, _ = x.shape
    _, n = w_mat.shape
    per = m // N_DEV
    nc = n // W

    def body(x_ref, w_ref, out_ref, part, sbuf, rbuf, gsend, obuf,
             s1, r1, s2, r2):
        my = lax.axis_index("i")

        part[...] = jnp.dot(x_ref[...], w_ref[...],
                            preferred_element_type=jnp.float32)
        for w in range(W):
            for k in range(1, N_DEV):
                p = lax.rem(my + k, N_DEV)
                sbuf[w, k] = part[pl.ds(p * per, per),
                                  w * nc:(w + 1) * nc].astype(jnp.bfloat16)

        barrier = pltpu.get_barrier_semaphore()
        for k in range(1, N_DEV):
            p = lax.rem(my + k, N_DEV)
            pl.semaphore_signal(barrier, inc=1, device_id=(p,),
                                device_id_type=pl.DeviceIdType.MESH)
        pl.semaphore_wait(barrier, N_DEV - 1)

        sends = []
        for w in range(W):
            for k in range(1, N_DEV):
                p = lax.rem(my + k, N_DEV)
                rdma = pltpu.make_async_remote_copy(
                    src_ref=sbuf.at[w, k], dst_ref=rbuf.at[w, N_DEV - k],
                    send_sem=s1.at[w, k], recv_sem=r1.at[w, N_DEV - k],
                    device_id=(p,), device_id_type=pl.DeviceIdType.MESH)
                rdma.start()
                sends.append(rdma)

        for w in range(W):
            cols = slice(w * nc, (w + 1) * nc)
            tot = part[pl.ds(my * per, per), cols]
            for j in range(1, N_DEV):
                recv = pltpu.make_async_remote_copy(
                    src_ref=sbuf.at[w, j], dst_ref=rbuf.at[w, j],
                    send_sem=s1.at[w, j], recv_sem=r1.at[w, j],
                    device_id=(my,), device_id_type=pl.DeviceIdType.MESH)
                recv.wait_recv()
                tot = tot + rbuf[w, j].astype(jnp.float32)

            g = _gelu(tot)
            out_ref[pl.ds(my * per, per), cols] = g
            gsend[w] = g.astype(jnp.bfloat16)

            for k in range(1, N_DEV):
                p = lax.rem(my + k, N_DEV)
                rdma = pltpu.make_async_remote_copy(
                    src_ref=gsend.at[w], dst_ref=obuf.at[w, N_DEV - k],
                    send_sem=s2.at[w, k], recv_sem=r2.at[w, N_DEV - k],
                    device_id=(p,), device_id_type=pl.DeviceIdType.MESH)
                rdma.start()
                sends.append(rdma)

        for w in range(W):
            cols = slice(w * nc, (w + 1) * nc)
            for j in range(1, N_DEV):
                recv = pltpu.make_async_remote_copy(
                    src_ref=gsend.at[w], dst_ref=obuf.at[w, j],
                    send_sem=s2.at[w, j], recv_sem=r2.at[w, j],
                    device_id=(my,), device_id_type=pl.DeviceIdType.MESH)
                recv.wait_recv()
                c = lax.rem(my + j, N_DEV)
                out_ref[pl.ds(c * per, per), cols] = obuf[w, j].astype(jnp.float32)

        for rdma in sends:
            rdma.wait_send()

    out_shape = jax.ShapeDtypeStruct((m, n), jnp.float32)
    return pl.pallas_call(
        body,
        out_shape=out_shape,
        in_specs=[pl.BlockSpec(memory_space=pltpu.VMEM),
                  pl.BlockSpec(memory_space=pltpu.VMEM)],
        out_specs=pl.BlockSpec(memory_space=pltpu.VMEM),
        scratch_shapes=[
            pltpu.VMEM((m, n), jnp.float32),
            pltpu.VMEM((W, N_DEV, per, nc), jnp.bfloat16),
            pltpu.VMEM((W, N_DEV, per, nc), jnp.bfloat16),
            pltpu.VMEM((W, per, nc), jnp.bfloat16),
            pltpu.VMEM((W, N_DEV, per, nc), jnp.bfloat16),
            pltpu.SemaphoreType.DMA((W, N_DEV)),
            pltpu.SemaphoreType.DMA((W, N_DEV)),
            pltpu.SemaphoreType.DMA((W, N_DEV)),
            pltpu.SemaphoreType.DMA((W, N_DEV)),
        ],
        compiler_params=pltpu.CompilerParams(collective_id=0),
    )(x, w_mat)


# device time: 58146 ns/iter; 2.8947x vs baseline; 1.0767x over previous
import jax
import jax.numpy as jnp
from jax import lax
from jax.experimental import pallas as pl
from jax.experimental.pallas import tpu as pltpu

N_DEV = 32
G = N_DEV // 2
W = 2


def _gelu(y):
    c = 0.7978845608028654
    return 0.5 * y * (1.0 + jnp.tanh(c * (y + 0.044715 * y * y * y)))


def kernel(x, w_mat):
    m, _ = x.shape
    _, n = w_mat.shape
    per = m // N_DEV
    nc = n // W

    def body(x_ref, w_ref, out_ref, part, xs, xr, cmb, s1b, r1b, gsend,
             fbuf, pbuf, sx, rx, s1, r1, s2, r2, sf, rf):
        my = lax.axis_index("i")
        q = lax.rem(my, 2)
        partner = my + 1 - 2 * q
        base = my - q

        def o_mine(e):
            return lax.rem(base + 2 * e, N_DEV) + q

        def o_partner(e):
            return lax.rem(base + 2 * e, N_DEV) + (1 - q)

        part[...] = jnp.dot(x_ref[...], w_ref[...],
                            preferred_element_type=jnp.float32)
        for w in range(W):
            cols = slice(w * nc, (w + 1) * nc)
            for e in range(G):
                xs[w, e] = part[pl.ds(o_partner(e) * per, per),
                                cols].astype(jnp.bfloat16)

        barrier = pltpu.get_barrier_semaphore()
        pl.semaphore_signal(barrier, inc=1, device_id=(partner,),
                            device_id_type=pl.DeviceIdType.MESH)
        for d in range(1, G):
            pl.semaphore_signal(barrier, inc=1, device_id=(o_mine(d),),
                                device_id_type=pl.DeviceIdType.MESH)
        pl.semaphore_wait(barrier, G)

        sends = []

        for w in range(W):
            rdma = pltpu.make_async_remote_copy(
                src_ref=xs.at[w], dst_ref=xr.at[w],
                send_sem=sx.at[w], recv_sem=rx.at[w],
                device_id=(partner,), device_id_type=pl.DeviceIdType.MESH)
            rdma.start()
            sends.append(rdma)

        for w in range(W):
            cols = slice(w * nc, (w + 1) * nc)
            recv = pltpu.make_async_remote_copy(
                src_ref=xs.at[w], dst_ref=xr.at[w],
                send_sem=sx.at[w], recv_sem=rx.at[w],
                device_id=(my,), device_id_type=pl.DeviceIdType.MESH)
            recv.wait_recv()
            for e in range(G):
                cmb[w, e] = (part[pl.ds(o_mine(e) * per, per), cols]
                             + xr[w, e].astype(jnp.float32))
            for d in range(1, G):
                s1b[w, d] = cmb[w, d].astype(jnp.bfloat16)
                rdma = pltpu.make_async_remote_copy(
                    src_ref=s1b.at[w, d], dst_ref=r1b.at[w, G - d],
                    send_sem=s1.at[w, d], recv_sem=r1.at[w, G - d],
                    device_id=(o_mine(d),),
                    device_id_type=pl.DeviceIdType.MESH)
                rdma.start()
                sends.append(rdma)

        for w in range(W):
            cols = slice(w * nc, (w + 1) * nc)
            tot = cmb[w, 0]
            for j in range(1, G):
                recv = pltpu.make_async_remote_copy(
                    src_ref=s1b.at[w, j], dst_ref=r1b.at[w, j],
                    send_sem=s1.at[w, j], recv_sem=r1.at[w, j],
                    device_id=(my,), device_id_type=pl.DeviceIdType.MESH)
                recv.wait_recv()
                tot = tot + r1b[w, j].astype(jnp.float32)
            g = _gelu(tot)
            out_ref[pl.ds(my * per, per), cols] = g
            gsend[w] = g.astype(jnp.bfloat16)
            fbuf[w, 0] = gsend[w]
            for d in range(1, G):
                rdma = pltpu.make_async_remote_copy(
                    src_ref=gsend.at[w], dst_ref=fbuf.at[w, G - d],
                    send_sem=s2.at[w, d], recv_sem=r2.at[w, G - d],
                    device_id=(o_mine(d),),
                    device_id_type=pl.DeviceIdType.MESH)
                rdma.start()
                sends.append(rdma)

        for w in range(W):
            cols = slice(w * nc, (w + 1) * nc)
            for j in range(1, G):
                recv = pltpu.make_async_remote_copy(
                    src_ref=gsend.at[w], dst_ref=fbuf.at[w, j],
                    send_sem=s2.at[w, j], recv_sem=r2.at[w, j],
                    device_id=(my,), device_id_type=pl.DeviceIdType.MESH)
                recv.wait_recv()
                out_ref[pl.ds(o_mine(j) * per, per), cols] = (
                    fbuf[w, j].astype(jnp.float32))
            rdma = pltpu.make_async_remote_copy(
                src_ref=fbuf.at[w], dst_ref=pbuf.at[w],
                send_sem=sf.at[w], recv_sem=rf.at[w],
                device_id=(partner,), device_id_type=pl.DeviceIdType.MESH)
            rdma.start()
            sends.append(rdma)

        for w in range(W):
            cols = slice(w * nc, (w + 1) * nc)
            recv = pltpu.make_async_remote_copy(
                src_ref=fbuf.at[w], dst_ref=pbuf.at[w],
                send_sem=sf.at[w], recv_sem=rf.at[w],
                device_id=(my,), device_id_type=pl.DeviceIdType.MESH)
            recv.wait_recv()
            for e in range(G):
                out_ref[pl.ds(o_partner(e) * per, per), cols] = (
                    pbuf[w, e].astype(jnp.float32))

        for rdma in sends:
            rdma.wait_send()

    out_shape = jax.ShapeDtypeStruct((m, n), jnp.float32)
    return pl.pallas_call(
        body,
        out_shape=out_shape,
        in_specs=[pl.BlockSpec(memory_space=pltpu.VMEM),
                  pl.BlockSpec(memory_space=pltpu.VMEM)],
        out_specs=pl.BlockSpec(memory_space=pltpu.VMEM),
        scratch_shapes=[
            pltpu.VMEM((m, n), jnp.float32),
            pltpu.VMEM((W, G, per, nc), jnp.bfloat16),
            pltpu.VMEM((W, G, per, nc), jnp.bfloat16),
            pltpu.VMEM((W, G, per, nc), jnp.float32),
            pltpu.VMEM((W, G, per, nc), jnp.bfloat16),
            pltpu.VMEM((W, G, per, nc), jnp.bfloat16),
            pltpu.VMEM((W, per, nc), jnp.bfloat16),
            pltpu.VMEM((W, G, per, nc), jnp.bfloat16),
            pltpu.VMEM((W, G, per, nc), jnp.bfloat16),
            pltpu.SemaphoreType.DMA((W,)),
            pltpu.SemaphoreType.DMA((W,)),
            pltpu.SemaphoreType.DMA((W, G)),
            pltpu.SemaphoreType.DMA((W, G)),
            pltpu.SemaphoreType.DMA((W, G)),
            pltpu.SemaphoreType.DMA((W, G)),
            pltpu.SemaphoreType.DMA((W,)),
            pltpu.SemaphoreType.DMA((W,)),
        ],
        compiler_params=pltpu.CompilerParams(collective_id=0),
    )(x, w_mat)
